# Initial kernel scaffold; baseline (speedup 1.0000x reference)
import jax
import jax.numpy as jnp
from jax.experimental import pallas as pl


def kernel(x, W):
    raise NotImplementedError("write your pallas kernel here")


# speedup vs baseline: 1.0263x; 1.0263x over previous
import functools

import jax
import jax.numpy as jnp
from jax import lax
from jax.experimental import pallas as pl
from jax.experimental.pallas import tpu as pltpu

N_ROWS = 16384
N_CODES = 8192
DIM = 32

R_BLK = 1024
K_BLK = 1024
N_R = N_ROWS // R_BLK
N_K = N_CODES // K_BLK

_LOSS_SCALE = 1.5 / float(N_ROWS * DIM)
_BIG_I32 = 2**30


def _tc_body(x_ref, wt_ref, idx_ref, loss_ref, best_ref, bidx_ref):
    i = pl.program_id(0)
    k = pl.program_id(1)

    xb = x_ref[...]
    wt = wt_ref[...]
    ab = lax.dot_general(xb, wt, (((1,), (0,)), ((), ())),
                         preferred_element_type=jnp.float32)
    a = jnp.sum(xb * xb, axis=1, keepdims=True)
    b = jnp.sum(wt * wt, axis=0, keepdims=True)
    d = (a + b) - 2.0 * ab

    cmin = jnp.min(d, axis=1, keepdims=True)
    iota = lax.broadcasted_iota(jnp.int32, (R_BLK, K_BLK), 1) + k * K_BLK
    cidx = jnp.min(jnp.where(d == cmin, iota, _BIG_I32),
                   axis=1, keepdims=True)

    @pl.when(k == 0)
    def _():
        best_ref[...] = cmin
        bidx_ref[...] = cidx

    @pl.when(k > 0)
    def _():
        prev = best_ref[...]
        upd = cmin < prev
        best_ref[...] = jnp.where(upd, cmin, prev)
        bidx_ref[...] = jnp.where(upd, cidx, bidx_ref[...])

    @pl.when(k == N_K - 1)
    def _():
        idx_ref[0] = bidx_ref[...]
        partial = jnp.sum(best_ref[...])

        @pl.when(i == 0)
        def _():
            loss_ref[0, 0] = partial

        @pl.when(i > 0)
        def _():
            loss_ref[0, 0] = loss_ref[0, 0] + partial

        @pl.when(i == N_R - 1)
        def _():
            loss_ref[0, 0] = loss_ref[0, 0] * _LOSS_SCALE


@functools.lru_cache(maxsize=1)
def _build_tc():
    return pl.pallas_call(
        _tc_body,
        grid=(N_R, N_K),
        in_specs=[
            pl.BlockSpec((R_BLK, DIM), lambda i, k: (i, 0)),
            pl.BlockSpec((DIM, K_BLK), lambda i, k: (0, k)),
        ],
        out_specs=[
            pl.BlockSpec((1, R_BLK, 1), lambda i, k: (i, 0, 0)),
            pl.BlockSpec((1, 1), lambda i, k: (0, 0),
                         memory_space=pltpu.SMEM),
        ],
        out_shape=[
            jax.ShapeDtypeStruct((N_R, R_BLK, 1), jnp.int32),
            jax.ShapeDtypeStruct((1, 1), jnp.float32),
        ],
        scratch_shapes=[
            pltpu.VMEM((R_BLK, 1), jnp.float32),
            pltpu.VMEM((R_BLK, 1), jnp.int32),
        ],
        compiler_params=pltpu.CompilerParams(
            dimension_semantics=("arbitrary", "arbitrary"),
        ),
    )


@functools.lru_cache(maxsize=1)
def _build_sc_gather():
    from jax.experimental.pallas import tpu_sc as plsc

    info = plsc.get_sparse_core_info()
    nc, ns = info.num_cores, info.num_subcores
    nw = nc * ns
    b_per_w = N_ROWS // nw
    n_chunks = b_per_w // 128

    mesh = plsc.VectorSubcoreMesh(core_axis_name="c", subcore_axis_name="s")

    @functools.partial(
        pl.kernel,
        mesh=mesh,
        out_type=jax.ShapeDtypeStruct((N_ROWS, DIM), jnp.float32),
        scratch_types=[
            pltpu.VMEM((n_chunks, 128), jnp.int32),
            pltpu.VMEM((b_per_w, DIM), jnp.float32),
            pltpu.SemaphoreType.DMA,
        ],
        compiler_params=pltpu.CompilerParams(use_tc_tiling_on_sc=False),
    )
    def sc_gather(table_hbm, idx_hbm, out_hbm, idx_v, rows_v, sem):
        wid = lax.axis_index("s") * nc + lax.axis_index("c")
        base = wid * b_per_w
        pltpu.sync_copy(idx_hbm.at[wid], idx_v)
        copies = []
        for j in range(n_chunks):
            copies.append(pltpu.async_copy(
                table_hbm.at[idx_v.at[j]],
                rows_v.at[pl.ds(j * 128, 128)],
                sem))
        for c in copies:
            c.wait()
        pltpu.sync_copy(rows_v, out_hbm.at[pl.ds(base, b_per_w)])

    def run(table, idx_flat):
        nonlocal_shape = (nw, n_chunks, 128)
        return sc_gather(table, idx_flat.reshape(nonlocal_shape))

    return run


def _gather(W, idx_flat):
    return _build_sc_gather()(W, idx_flat)


def kernel(x, W):
    bsz, seq, dim = x.shape
    xf = x.reshape(bsz * seq, dim)
    idx3, loss11 = _build_tc()(xf, W.T)
    idx_flat = idx3.reshape(N_ROWS)
    q = _gather(W, idx_flat)
    out = (xf + (q - xf)).reshape(bsz, seq, dim)
    loss = loss11[0, 0]
    return (out, loss)
